# baseline (speedup 1.0000x reference)
import functools

import jax
import jax.numpy as jnp
from jax import lax
from jax.experimental import pallas as pl
from jax.experimental.pallas import tpu as pltpu
from jax.experimental.pallas import tpu_sc as plsc

_N, _H, _E, _K, _A, _NB = 8192, 1024, 8, 2, 7, 256
_O = _A * _NB
_H2 = 2 * _H
_T = 256
_MAXT = (2 * _N) // _T + _E
_S = _MAXT * _T

_NW = 32
_SC_NC = 2
_APW = 2 * _N // _NW
_NCH = _APW // 16
_TPW = _N // _NW
_CCH = 32
_NCC = _TPW // _CCH


def _router_body(x_ref, wr_ref, br_ref, i1_ref, g1_ref, g2_ref,
                 r1_ref, cnt_ref, carry):
    i = pl.program_id(0)

    @pl.when(i == 0)
    def _():
        carry[...] = jnp.zeros_like(carry)

    logits = jnp.dot(x_ref[...], wr_ref[...],
                     preferred_element_type=jnp.float32) + br_ref[...]
    bt = logits.shape[0]
    lane = lax.broadcasted_iota(jnp.int32, (bt, _E), 1)
    m1 = jnp.max(logits, axis=1, keepdims=True)
    idx1 = jnp.min(jnp.where(logits == m1, lane, _E), axis=1, keepdims=True)
    masked = jnp.where(lane == idx1, -jnp.inf, logits)
    m2 = jnp.max(masked, axis=1, keepdims=True)
    idx2 = jnp.min(jnp.where(masked == m2, lane, _E), axis=1, keepdims=True)
    d = jnp.exp(m2 - m1)
    g1 = 1.0 / (1.0 + d)
    oh = jnp.where(lane == idx1, 1.0, 0.0) + jnp.where(lane == idx2, 1.0, 0.0)
    row = lax.broadcasted_iota(jnp.int32, (bt, bt), 0)
    col = lax.broadcasted_iota(jnp.int32, (bt, bt), 1)
    tri = jnp.where(row > col, 1.0, 0.0)
    pfx = jnp.dot(tri, oh, preferred_element_type=jnp.float32) + carry[...]
    r1 = jnp.sum(jnp.where(lane == idx1, pfx, 0.0), axis=1, keepdims=True)
    r2 = jnp.sum(jnp.where(lane == idx2, pfx, 0.0), axis=1, keepdims=True)
    i1_ref[...] = jnp.concatenate([idx1, idx2], axis=1)
    g1_ref[...] = g1
    g2_ref[...] = 1.0 - g1
    r1_ref[...] = jnp.concatenate([r1, r2], axis=1).astype(jnp.int32)
    newc = carry[...] + jnp.sum(oh, axis=0, keepdims=True)
    carry[...] = newc

    @pl.when(i == pl.num_programs(0) - 1)
    def _():
        cnt_ref[...] = newc.astype(jnp.int32)


def _router(x, Wr, br):
    bt = 1024
    n2 = jax.ShapeDtypeStruct((_N, 2), jnp.int32)
    nf = jax.ShapeDtypeStruct((_N, 1), jnp.float32)
    return pl.pallas_call(
        _router_body,
        grid=(_N // bt,),
        in_specs=[
            pl.BlockSpec((bt, _H), lambda i: (i, 0)),
            pl.BlockSpec((_H, _E), lambda i: (0, 0)),
            pl.BlockSpec((1, _E), lambda i: (0, 0)),
        ],
        out_specs=[pl.BlockSpec((bt, 2), lambda i: (i, 0)),
                   pl.BlockSpec((bt, 1), lambda i: (i, 0)),
                   pl.BlockSpec((bt, 1), lambda i: (i, 0)),
                   pl.BlockSpec((bt, 2), lambda i: (i, 0)),
                   pl.BlockSpec((1, _E), lambda i: (0, 0))],
        out_shape=[n2, nf, nf, n2,
                   jax.ShapeDtypeStruct((1, _E), jnp.int32)],
        scratch_shapes=[pltpu.VMEM((1, _E), jnp.float32)],
    )(x, Wr, br.reshape(1, _E))


def _dispatch(x, eidx_flat, rank_flat, offs16):
    mesh = plsc.VectorSubcoreMesh(core_axis_name="c", subcore_axis_name="s")

    @functools.partial(
        pl.kernel,
        out_type=[jax.ShapeDtypeStruct((_S, _H), jnp.float32),
                  jax.ShapeDtypeStruct((_N // 16, 16), jnp.int32),
                  jax.ShapeDtypeStruct((_N // 16, 16), jnp.int32)],
        mesh=mesh,
        scratch_types=[
            pltpu.VMEM((_APW,), jnp.int32),
            pltpu.VMEM((_APW,), jnp.int32),
            pltpu.VMEM((16,), jnp.int32),
            pltpu.VMEM((16, 16), jnp.int32),
            pltpu.VMEM((16, 16), jnp.int32),
            pltpu.VMEM((32, _H), jnp.float32),
            pltpu.VMEM((32, _H), jnp.float32),
            pltpu.SemaphoreType.DMA,
            pltpu.SemaphoreType.DMA,
        ],
        compiler_params=pltpu.CompilerParams(needs_layout_passes=False),
    )
    def k(x_hbm, e_hbm, r_hbm, o_hbm, xs_hbm, p0_hbm, p1_hbm,
          ev, rv, ov, p0b, p1b, xb0, xb1, sld, ssc):
        wid = lax.axis_index("s") * _SC_NC + lax.axis_index("c")
        abase = wid * _APW
        tbase = wid * _TPW
        pltpu.sync_copy(e_hbm.at[pl.ds(abase, _APW)], ev)
        pltpu.sync_copy(r_hbm.at[pl.ds(abase, _APW)], rv)
        pltpu.sync_copy(o_hbm, ov)
        lanes2 = lax.iota(jnp.int32, 16) * 2
        for r in range(16):
            i0 = r * 32 + lanes2
            e0 = plsc.load_gather(ev, [i0])
            r0 = plsc.load_gather(rv, [i0])
            e1 = plsc.load_gather(ev, [i0 + 1])
            r1 = plsc.load_gather(rv, [i0 + 1])
            p0b[r, :] = plsc.load_gather(ov, [e0]) + r0
            p1b[r, :] = plsc.load_gather(ov, [e1]) + r1
        xbufs = (xb0, xb1)
        nch = 8
        lds = [None] * nch
        scs = [[] for _ in range(nch)]
        lds[0] = pltpu.async_copy(x_hbm.at[pl.ds(tbase, 32), :], xb0, sld)
        for c in range(nch):
            lds[c].wait()
            if c + 1 < nch:
                for h in scs[c - 1] if c >= 1 else ():
                    h.wait()
                lds[c + 1] = pltpu.async_copy(
                    x_hbm.at[pl.ds(tbase + (c + 1) * 32, 32), :],
                    xbufs[(c + 1) % 2], sld)
            xb = xbufs[c % 2]
            for half in range(2):
                rrow = 2 * c + half
                src = xb.at[pl.ds(half * 16, 16), :]
                scs[c].append(pltpu.async_copy(src, xs_hbm.at[p0b.at[rrow]], ssc))
                scs[c].append(pltpu.async_copy(src, xs_hbm.at[p1b.at[rrow]], ssc))
        for h in scs[nch - 2] + scs[nch - 1]:
            h.wait()
        pltpu.sync_copy(p0b, p0_hbm.at[pl.ds(wid * 16, 16), :])
        pltpu.sync_copy(p1b, p1_hbm.at[pl.ds(wid * 16, 16), :])

    return k(x, eidx_flat, rank_flat, offs16)


def _gffn_body(eot_ref, x_ref, w1_ref, b1_ref, w2_ref, b2_ref, o_ref):
    h = jnp.dot(x_ref[...].astype(jnp.bfloat16), w1_ref[0],
                preferred_element_type=jnp.float32)
    h = h + b1_ref[0]
    h = 0.5 * h * (1.0 + lax.erf(h * 0.7071067811865476))
    o_ref[...] = jnp.dot(h.astype(jnp.bfloat16), w2_ref[0],
                         preferred_element_type=jnp.float32) + b2_ref[0]


def _grouped_ffn(eot, xs, W1, b1, W2, b2):
    grid_spec = pltpu.PrefetchScalarGridSpec(
        num_scalar_prefetch=1,
        grid=(_MAXT,),
        in_specs=[
            pl.BlockSpec((_T, _H), lambda i, eot_ref: (i, 0)),
            pl.BlockSpec((1, _H, _H2), lambda i, eot_ref: (eot_ref[i], 0, 0)),
            pl.BlockSpec((1, 1, _H2), lambda i, eot_ref: (eot_ref[i], 0, 0)),
            pl.BlockSpec((1, _H2, _O), lambda i, eot_ref: (eot_ref[i], 0, 0)),
            pl.BlockSpec((1, 1, _O), lambda i, eot_ref: (eot_ref[i], 0, 0)),
        ],
        out_specs=pl.BlockSpec((_T, _O), lambda i, eot_ref: (i, 0)),
    )
    return pl.pallas_call(
        _gffn_body,
        grid_spec=grid_spec,
        out_shape=jax.ShapeDtypeStruct((_S, _O), jnp.float32),
        compiler_params=pltpu.CompilerParams(
            dimension_semantics=("arbitrary",),
            vmem_limit_bytes=120 * 1024 * 1024,
        ),
    )(eot, xs, W1.astype(jnp.bfloat16), b1.reshape(_E, 1, _H2),
      W2.astype(jnp.bfloat16), b2.reshape(_E, 1, _O))


def _combine_gather(y, pos0, pos1):
    mesh = plsc.VectorSubcoreMesh(core_axis_name="c", subcore_axis_name="s")

    @functools.partial(
        pl.kernel,
        out_type=[jax.ShapeDtypeStruct((_N, _O), jnp.float32),
                  jax.ShapeDtypeStruct((_N, _O), jnp.float32)],
        mesh=mesh,
        scratch_types=[
            pltpu.VMEM((16, 16), jnp.int32),
            pltpu.VMEM((16, 16), jnp.int32),
            pltpu.VMEM((16, _O), jnp.float32),
            pltpu.VMEM((16, _O), jnp.float32),
            pltpu.VMEM((16, _O), jnp.float32),
            pltpu.VMEM((16, _O), jnp.float32),
            pltpu.SemaphoreType.DMA,
            pltpu.SemaphoreType.DMA,
        ],
    )
    def k(y_hbm, p0_hbm, p1_hbm, ya_hbm, yb_hbm,
          p0v, p1v, a0, a1, b0, b1, sg, sw):
        wid = lax.axis_index("s") * _SC_NC + lax.axis_index("c")
        tbase = wid * _TPW
        pltpu.sync_copy(p0_hbm.at[pl.ds(wid * 16, 16), :], p0v)
        pltpu.sync_copy(p1_hbm.at[pl.ds(wid * 16, 16), :], p1v)
        A = (a0, a1)
        B = (b0, b1)
        nch = 16
        ga = [None] * nch
        gb = [None] * nch
        wa = [None] * nch
        wb = [None] * nch
        ga[0] = pltpu.async_copy(y_hbm.at[p0v.at[0]], a0, sg)
        gb[0] = pltpu.async_copy(y_hbm.at[p1v.at[0]], b0, sg)
        for c in range(nch):
            ga[c].wait()
            gb[c].wait()
            if c + 1 < nch:
                if c >= 1:
                    wa[c - 1].wait()
                    wb[c - 1].wait()
                ga[c + 1] = pltpu.async_copy(y_hbm.at[p0v.at[c + 1]],
                                             A[(c + 1) % 2], sg)
                gb[c + 1] = pltpu.async_copy(y_hbm.at[p1v.at[c + 1]],
                                             B[(c + 1) % 2], sg)
            dst = pl.ds(tbase + c * 16, 16)
            wa[c] = pltpu.async_copy(A[c % 2], ya_hbm.at[dst, :], sw)
            wb[c] = pltpu.async_copy(B[c % 2], yb_hbm.at[dst, :], sw)
        wa[nch - 2].wait()
        wb[nch - 2].wait()
        wa[nch - 1].wait()
        wb[nch - 1].wait()

    return k(y, pos0, pos1)


def _final_body(ya_ref, yb_ref, g1_ref, g2_ref, o_ref):
    o_ref[...] = g1_ref[...] * ya_ref[...] + g2_ref[...] * yb_ref[...]


def _final_combine(ya, yb, g1, g2):
    bt = 512
    return pl.pallas_call(
        _final_body,
        grid=(_N // bt,),
        in_specs=[
            pl.BlockSpec((bt, _O), lambda i: (i, 0)),
            pl.BlockSpec((bt, _O), lambda i: (i, 0)),
            pl.BlockSpec((bt, 1), lambda i: (i, 0)),
            pl.BlockSpec((bt, 1), lambda i: (i, 0)),
        ],
        out_specs=pl.BlockSpec((bt, _O), lambda i: (i, 0)),
        out_shape=jax.ShapeDtypeStruct((_N, _O), jnp.float32),
    )(ya, yb, g1, g2)


def kernel(hidden_states, Wr, br, W1, b1, W2, b2):
    eidx2, g1, g2, rank2, cnt = _router(hidden_states, Wr, br)
    counts = cnt.reshape(_E)
    tile_counts = (counts + _T - 1) // _T
    cum = jnp.cumsum(tile_counts)
    off_rows = (cum - tile_counts) * _T
    eot = jnp.minimum(
        jnp.sum((jnp.arange(_MAXT, dtype=jnp.int32)[:, None]
                 >= cum[None, :]).astype(jnp.int32), axis=1),
        _E - 1).astype(jnp.int32)
    offs16 = jnp.concatenate([off_rows, jnp.zeros((8,), jnp.int32)])
    xs, p0, p1 = _dispatch(hidden_states, eidx2.reshape(2 * _N),
                           rank2.reshape(2 * _N), offs16)
    y = _grouped_ffn(eot, xs, W1, b1, W2, b2)
    ya, yb = _combine_gather(y, p0, p1)
    out = _final_combine(ya, yb, g1, g2)
    return out.reshape(_N, _A, _NB)


# speedup vs baseline: 1.0857x; 1.0857x over previous
import functools

import jax
import jax.numpy as jnp
from jax import lax
from jax.experimental import pallas as pl
from jax.experimental.pallas import tpu as pltpu
from jax.experimental.pallas import tpu_sc as plsc

_N, _H, _E, _K, _A, _NB = 8192, 1024, 8, 2, 7, 256
_O = _A * _NB
_H2 = 2 * _H
_T = 256
_MAXT = (2 * _N) // _T + _E
_S = _MAXT * _T

_NW = 32
_SC_NC = 2
_APW = 2 * _N // _NW
_NCH = _APW // 16
_TPW = _N // _NW
_CCH = 32
_NCC = _TPW // _CCH


def _router_body(x_ref, wr_ref, br_ref, i1_ref, g1_ref, g2_ref,
                 r1_ref, cnt_ref, carry):
    i = pl.program_id(0)

    @pl.when(i == 0)
    def _():
        carry[...] = jnp.zeros_like(carry)

    logits = jnp.dot(x_ref[...], wr_ref[...],
                     preferred_element_type=jnp.float32) + br_ref[...]
    bt = logits.shape[0]
    lane = lax.broadcasted_iota(jnp.int32, (bt, _E), 1)
    m1 = jnp.max(logits, axis=1, keepdims=True)
    idx1 = jnp.min(jnp.where(logits == m1, lane, _E), axis=1, keepdims=True)
    masked = jnp.where(lane == idx1, -jnp.inf, logits)
    m2 = jnp.max(masked, axis=1, keepdims=True)
    idx2 = jnp.min(jnp.where(masked == m2, lane, _E), axis=1, keepdims=True)
    d = jnp.exp(m2 - m1)
    g1 = 1.0 / (1.0 + d)
    oh = jnp.where(lane == idx1, 1.0, 0.0) + jnp.where(lane == idx2, 1.0, 0.0)
    row = lax.broadcasted_iota(jnp.int32, (bt, bt), 0)
    col = lax.broadcasted_iota(jnp.int32, (bt, bt), 1)
    tri = jnp.where(row > col, 1.0, 0.0)
    pfx = jnp.dot(tri, oh, preferred_element_type=jnp.float32) + carry[...]
    r1 = jnp.sum(jnp.where(lane == idx1, pfx, 0.0), axis=1, keepdims=True)
    r2 = jnp.sum(jnp.where(lane == idx2, pfx, 0.0), axis=1, keepdims=True)
    i1_ref[...] = jnp.concatenate([idx1, idx2], axis=1)
    g1_ref[...] = g1
    g2_ref[...] = 1.0 - g1
    r1_ref[...] = jnp.concatenate([r1, r2], axis=1).astype(jnp.int32)
    newc = carry[...] + jnp.sum(oh, axis=0, keepdims=True)
    carry[...] = newc

    @pl.when(i == pl.num_programs(0) - 1)
    def _():
        cnt_ref[...] = newc.astype(jnp.int32)


def _router(x, Wr, br):
    bt = 1024
    n2 = jax.ShapeDtypeStruct((_N, 2), jnp.int32)
    nf = jax.ShapeDtypeStruct((_N, 1), jnp.float32)
    return pl.pallas_call(
        _router_body,
        grid=(_N // bt,),
        in_specs=[
            pl.BlockSpec((bt, _H), lambda i: (i, 0)),
            pl.BlockSpec((_H, _E), lambda i: (0, 0)),
            pl.BlockSpec((1, _E), lambda i: (0, 0)),
        ],
        out_specs=[pl.BlockSpec((bt, 2), lambda i: (i, 0)),
                   pl.BlockSpec((bt, 1), lambda i: (i, 0)),
                   pl.BlockSpec((bt, 1), lambda i: (i, 0)),
                   pl.BlockSpec((bt, 2), lambda i: (i, 0)),
                   pl.BlockSpec((1, _E), lambda i: (0, 0))],
        out_shape=[n2, nf, nf, n2,
                   jax.ShapeDtypeStruct((1, _E), jnp.int32)],
        scratch_shapes=[pltpu.VMEM((1, _E), jnp.float32)],
    )(x, Wr, br.reshape(1, _E))


def _dispatch(x, eidx_flat, rank_flat, offs16):
    mesh = plsc.VectorSubcoreMesh(core_axis_name="c", subcore_axis_name="s")

    @functools.partial(
        pl.kernel,
        out_type=[jax.ShapeDtypeStruct((_S, _H), jnp.float32),
                  jax.ShapeDtypeStruct((_N // 16, 16), jnp.int32),
                  jax.ShapeDtypeStruct((_N // 16, 16), jnp.int32)],
        mesh=mesh,
        scratch_types=[
            pltpu.VMEM((_APW,), jnp.int32),
            pltpu.VMEM((_APW,), jnp.int32),
            pltpu.VMEM((16,), jnp.int32),
            pltpu.VMEM((16, 16), jnp.int32),
            pltpu.VMEM((16, 16), jnp.int32),
            pltpu.VMEM((32, _H), jnp.float32),
            pltpu.VMEM((32, _H), jnp.float32),
            pltpu.SemaphoreType.DMA,
            pltpu.SemaphoreType.DMA,
        ],
        compiler_params=pltpu.CompilerParams(needs_layout_passes=False),
    )
    def k(x_hbm, e_hbm, r_hbm, o_hbm, xs_hbm, p0_hbm, p1_hbm,
          ev, rv, ov, p0b, p1b, xb0, xb1, sld, ssc):
        wid = lax.axis_index("s") * _SC_NC + lax.axis_index("c")
        abase = wid * _APW
        tbase = wid * _TPW
        pltpu.sync_copy(e_hbm.at[pl.ds(abase, _APW)], ev)
        pltpu.sync_copy(r_hbm.at[pl.ds(abase, _APW)], rv)
        pltpu.sync_copy(o_hbm, ov)
        lanes2 = lax.iota(jnp.int32, 16) * 2
        for r in range(16):
            i0 = r * 32 + lanes2
            e0 = plsc.load_gather(ev, [i0])
            r0 = plsc.load_gather(rv, [i0])
            e1 = plsc.load_gather(ev, [i0 + 1])
            r1 = plsc.load_gather(rv, [i0 + 1])
            p0b[r, :] = plsc.load_gather(ov, [e0]) + r0
            p1b[r, :] = plsc.load_gather(ov, [e1]) + r1
        xbufs = (xb0, xb1)
        nch = 8
        lds = [None] * nch
        scs = [[] for _ in range(nch)]
        lds[0] = pltpu.async_copy(x_hbm.at[pl.ds(tbase, 32), :], xb0, sld)
        for c in range(nch):
            lds[c].wait()
            if c + 1 < nch:
                for h in scs[c - 1] if c >= 1 else ():
                    h.wait()
                lds[c + 1] = pltpu.async_copy(
                    x_hbm.at[pl.ds(tbase + (c + 1) * 32, 32), :],
                    xbufs[(c + 1) % 2], sld)
            xb = xbufs[c % 2]
            for half in range(2):
                rrow = 2 * c + half
                src = xb.at[pl.ds(half * 16, 16), :]
                scs[c].append(pltpu.async_copy(src, xs_hbm.at[p0b.at[rrow]], ssc))
                scs[c].append(pltpu.async_copy(src, xs_hbm.at[p1b.at[rrow]], ssc))
        for h in scs[nch - 2] + scs[nch - 1]:
            h.wait()
        pltpu.sync_copy(p0b, p0_hbm.at[pl.ds(wid * 16, 16), :])
        pltpu.sync_copy(p1b, p1_hbm.at[pl.ds(wid * 16, 16), :])

    return k(x, eidx_flat, rank_flat, offs16)


def _gffn_body(eot_ref, x_ref, w1_ref, b1_ref, w2_ref, b2_ref, o_ref):
    h = jnp.dot(x_ref[...], w1_ref[0], preferred_element_type=jnp.float32)
    h = h + b1_ref[0]
    h = 0.5 * h * (1.0 + lax.erf(h * 0.7071067811865476))
    o_ref[...] = jnp.dot(h, w2_ref[0],
                         preferred_element_type=jnp.float32) + b2_ref[0]


def _grouped_ffn(eot, xs, W1, b1, W2, b2):
    grid_spec = pltpu.PrefetchScalarGridSpec(
        num_scalar_prefetch=1,
        grid=(_MAXT,),
        in_specs=[
            pl.BlockSpec((_T, _H), lambda i, eot_ref: (i, 0)),
            pl.BlockSpec((1, _H, _H2), lambda i, eot_ref: (eot_ref[i], 0, 0)),
            pl.BlockSpec((1, 1, _H2), lambda i, eot_ref: (eot_ref[i], 0, 0)),
            pl.BlockSpec((1, _H2, _O), lambda i, eot_ref: (eot_ref[i], 0, 0)),
            pl.BlockSpec((1, 1, _O), lambda i, eot_ref: (eot_ref[i], 0, 0)),
        ],
        out_specs=pl.BlockSpec((_T, _O), lambda i, eot_ref: (i, 0)),
    )
    return pl.pallas_call(
        _gffn_body,
        grid_spec=grid_spec,
        out_shape=jax.ShapeDtypeStruct((_S, _O), jnp.float32),
        compiler_params=pltpu.CompilerParams(
            dimension_semantics=("arbitrary",),
            vmem_limit_bytes=120 * 1024 * 1024,
        ),
    )(eot, xs, W1, b1.reshape(_E, 1, _H2), W2, b2.reshape(_E, 1, _O))


def _combine_gather(y, pos0, pos1):
    mesh = plsc.VectorSubcoreMesh(core_axis_name="c", subcore_axis_name="s")

    @functools.partial(
        pl.kernel,
        out_type=[jax.ShapeDtypeStruct((_N, _O), jnp.float32),
                  jax.ShapeDtypeStruct((_N, _O), jnp.float32)],
        mesh=mesh,
        scratch_types=[
            pltpu.VMEM((16, 16), jnp.int32),
            pltpu.VMEM((16, 16), jnp.int32),
            pltpu.VMEM((16, _O), jnp.float32),
            pltpu.VMEM((16, _O), jnp.float32),
            pltpu.VMEM((16, _O), jnp.float32),
            pltpu.VMEM((16, _O), jnp.float32),
            pltpu.SemaphoreType.DMA,
            pltpu.SemaphoreType.DMA,
        ],
    )
    def k(y_hbm, p0_hbm, p1_hbm, ya_hbm, yb_hbm,
          p0v, p1v, a0, a1, b0, b1, sg, sw):
        wid = lax.axis_index("s") * _SC_NC + lax.axis_index("c")
        tbase = wid * _TPW
        pltpu.sync_copy(p0_hbm.at[pl.ds(wid * 16, 16), :], p0v)
        pltpu.sync_copy(p1_hbm.at[pl.ds(wid * 16, 16), :], p1v)
        A = (a0, a1)
        B = (b0, b1)
        nch = 16
        ga = [None] * nch
        gb = [None] * nch
        wa = [None] * nch
        wb = [None] * nch
        ga[0] = pltpu.async_copy(y_hbm.at[p0v.at[0]], a0, sg)
        gb[0] = pltpu.async_copy(y_hbm.at[p1v.at[0]], b0, sg)
        for c in range(nch):
            ga[c].wait()
            gb[c].wait()
            if c + 1 < nch:
                if c >= 1:
                    wa[c - 1].wait()
                    wb[c - 1].wait()
                ga[c + 1] = pltpu.async_copy(y_hbm.at[p0v.at[c + 1]],
                                             A[(c + 1) % 2], sg)
                gb[c + 1] = pltpu.async_copy(y_hbm.at[p1v.at[c + 1]],
                                             B[(c + 1) % 2], sg)
            dst = pl.ds(tbase + c * 16, 16)
            wa[c] = pltpu.async_copy(A[c % 2], ya_hbm.at[dst, :], sw)
            wb[c] = pltpu.async_copy(B[c % 2], yb_hbm.at[dst, :], sw)
        wa[nch - 2].wait()
        wb[nch - 2].wait()
        wa[nch - 1].wait()
        wb[nch - 1].wait()

    return k(y, pos0, pos1)


def _final_body(ya_ref, yb_ref, g1_ref, g2_ref, o_ref):
    o_ref[...] = g1_ref[...] * ya_ref[...] + g2_ref[...] * yb_ref[...]


def _final_combine(ya, yb, g1, g2):
    bt = 512
    return pl.pallas_call(
        _final_body,
        grid=(_N // bt,),
        in_specs=[
            pl.BlockSpec((bt, _O), lambda i: (i, 0)),
            pl.BlockSpec((bt, _O), lambda i: (i, 0)),
            pl.BlockSpec((bt, 1), lambda i: (i, 0)),
            pl.BlockSpec((bt, 1), lambda i: (i, 0)),
        ],
        out_specs=pl.BlockSpec((bt, _O), lambda i: (i, 0)),
        out_shape=jax.ShapeDtypeStruct((_N, _O), jnp.float32),
    )(ya, yb, g1, g2)


def kernel(hidden_states, Wr, br, W1, b1, W2, b2):
    eidx2, g1, g2, rank2, cnt = _router(hidden_states, Wr, br)
    counts = cnt.reshape(_E)
    tile_counts = (counts + _T - 1) // _T
    cum = jnp.cumsum(tile_counts)
    off_rows = (cum - tile_counts) * _T
    eot = jnp.minimum(
        jnp.sum((jnp.arange(_MAXT, dtype=jnp.int32)[:, None]
                 >= cum[None, :]).astype(jnp.int32), axis=1),
        _E - 1).astype(jnp.int32)
    offs16 = jnp.concatenate([off_rows, jnp.zeros((8,), jnp.int32)])
    xs, p0, p1 = _dispatch(hidden_states, eidx2.reshape(2 * _N),
                           rank2.reshape(2 * _N), offs16)
    y = _grouped_ffn(eot, xs, W1, b1, W2, b2)
    ya, yb = _combine_gather(y, p0, p1)
    out = _final_combine(ya, yb, g1, g2)
    return out.reshape(_N, _A, _NB)
